# baseline (speedup 1.0000x reference)
import functools
import math

import jax
import jax.numpy as jnp
import numpy as np
from jax import lax
from jax.experimental import pallas as pl
from jax.experimental.pallas import tpu as pltpu
from jax.experimental.pallas import tpu_sc as plsc

_NUM_TIMESTEPS = 1000


def _schedule_tables():
    steps = _NUM_TIMESTEPS + 1
    x = np.linspace(0, _NUM_TIMESTEPS, steps, dtype=np.float64)
    s = 0.008
    alphas_cumprod = np.cos((x / _NUM_TIMESTEPS + s) / (1 + s) * math.pi * 0.5) ** 2
    alphas_cumprod = alphas_cumprod / alphas_cumprod[0]
    betas = np.clip(1 - alphas_cumprod[1:] / alphas_cumprod[:-1], 0, 0.999)
    ac = np.cumprod(1.0 - betas, axis=0)
    sqrt_ac = np.sqrt(ac).astype(np.float32)
    sqrt_om = np.sqrt(1.0 - ac).astype(np.float32)
    return sqrt_ac, sqrt_om


_SQRT_AC, _SQRT_OM = _schedule_tables()

_LANES = 16


def _sc_gather(B):
    mesh = plsc.VectorSubcoreMesh(core_axis_name="c", subcore_axis_name="s")

    @functools.partial(
        pl.kernel,
        mesh=mesh,
        out_type=[
            jax.ShapeDtypeStruct((B,), jnp.float32),
            jax.ShapeDtypeStruct((B,), jnp.float32),
        ],
        scratch_types=[
            pltpu.VMEM((B,), jnp.int32),
            pltpu.VMEM((_NUM_TIMESTEPS,), jnp.float32),
            pltpu.VMEM((_NUM_TIMESTEPS,), jnp.float32),
            pltpu.VMEM((B,), jnp.float32),
            pltpu.VMEM((B,), jnp.float32),
        ],
        compiler_params=pltpu.CompilerParams(needs_layout_passes=False),
    )
    def gather(ts_hbm, ta_hbm, tb_hbm, ca_hbm, cb_hbm, ts_v, ta_v, tb_v, ca_v, cb_v):
        wid = lax.axis_index("s") * 2 + lax.axis_index("c")

        @pl.when(wid == 0)
        def _():
            pltpu.sync_copy(ts_hbm, ts_v)
            pltpu.sync_copy(ta_hbm, ta_v)
            pltpu.sync_copy(tb_hbm, tb_v)
            for i in range(B // _LANES):
                idx = ts_v[pl.ds(i * _LANES, _LANES)]
                ca_v[pl.ds(i * _LANES, _LANES)] = plsc.load_gather(ta_v, [idx])
                cb_v[pl.ds(i * _LANES, _LANES)] = plsc.load_gather(tb_v, [idx])
            pltpu.sync_copy(ca_v, ca_hbm)
            pltpu.sync_copy(cb_v, cb_hbm)

    return gather


_NB = 8


def _tc_body(ca_ref, cb_ref, x_ref, n_ref, o_ref):
    g = pl.program_id(0)
    a = jnp.stack([ca_ref[g * _NB + j] for j in range(_NB)]).reshape(_NB, 1, 1)
    s = jnp.stack([cb_ref[g * _NB + j] for j in range(_NB)]).reshape(_NB, 1, 1)
    o_ref[...] = a * x_ref[...] + s * n_ref[...]


def kernel(x_start, noise, timesteps):
    B, C, H, W = x_start.shape
    R = C * H
    x3 = x_start.reshape(B, R, W)
    n3 = noise.reshape(B, R, W)
    ts = timesteps.astype(jnp.int32)

    ca, cb = _sc_gather(B)(ts, jnp.asarray(_SQRT_AC), jnp.asarray(_SQRT_OM))

    grid_spec = pltpu.PrefetchScalarGridSpec(
        num_scalar_prefetch=2,
        grid=(B // _NB,),
        in_specs=[
            pl.BlockSpec((_NB, R, W), lambda b, *_: (b, 0, 0)),
            pl.BlockSpec((_NB, R, W), lambda b, *_: (b, 0, 0)),
        ],
        out_specs=pl.BlockSpec((_NB, R, W), lambda b, *_: (b, 0, 0)),
    )
    out = pl.pallas_call(
        _tc_body,
        grid_spec=grid_spec,
        out_shape=jax.ShapeDtypeStruct((B, R, W), x_start.dtype),
    )(ca, cb, x3, n3)
    return out.reshape(B, C, H, W)


# speedup vs baseline: 1.3814x; 1.3814x over previous
import math

import jax
import jax.numpy as jnp
import numpy as np
from jax.experimental import pallas as pl
from jax.experimental.pallas import tpu as pltpu

_NUM_TIMESTEPS = 1000


def _schedule_tables():
    steps = _NUM_TIMESTEPS + 1
    x = np.linspace(0, _NUM_TIMESTEPS, steps, dtype=np.float64)
    s = 0.008
    alphas_cumprod = np.cos((x / _NUM_TIMESTEPS + s) / (1 + s) * math.pi * 0.5) ** 2
    alphas_cumprod = alphas_cumprod / alphas_cumprod[0]
    betas = np.clip(1 - alphas_cumprod[1:] / alphas_cumprod[:-1], 0, 0.999)
    ac = np.cumprod(1.0 - betas, axis=0)
    sqrt_ac = np.sqrt(ac).astype(np.float32)
    sqrt_om = np.sqrt(1.0 - ac).astype(np.float32)
    return sqrt_ac, sqrt_om


_SQRT_AC, _SQRT_OM = _schedule_tables()

_RING = 4


def _body(ts_ref, ta_ref, tb_ref, x_hbm, n_hbm, o_hbm,
          xb, nb, ob, rsem, wsem):
    B = x_hbm.shape[0]

    for b in range(_RING):
        s = b % _RING
        pltpu.make_async_copy(x_hbm.at[b], xb.at[s], rsem.at[2 * s]).start()
        pltpu.make_async_copy(n_hbm.at[b], nb.at[s], rsem.at[2 * s + 1]).start()

    for b in range(B):
        s = b % _RING
        pltpu.make_async_copy(x_hbm.at[b], xb.at[s], rsem.at[2 * s]).wait()
        pltpu.make_async_copy(n_hbm.at[b], nb.at[s], rsem.at[2 * s + 1]).wait()
        if b >= _RING:
            pltpu.make_async_copy(ob.at[s], o_hbm.at[b - _RING], wsem.at[s]).wait()
        t = ts_ref[b]
        a = ta_ref[t]
        c = tb_ref[t]
        ob[s] = a * xb[s] + c * nb[s]
        pltpu.make_async_copy(ob.at[s], o_hbm.at[b], wsem.at[s]).start()
        nxt = b + _RING
        if nxt < B:
            pltpu.make_async_copy(x_hbm.at[nxt], xb.at[s], rsem.at[2 * s]).start()
            pltpu.make_async_copy(n_hbm.at[nxt], nb.at[s], rsem.at[2 * s + 1]).start()

    for b in range(B - _RING, B):
        s = b % _RING
        pltpu.make_async_copy(ob.at[s], o_hbm.at[b], wsem.at[s]).wait()


def kernel(x_start, noise, timesteps):
    B, C, H, W = x_start.shape
    R = C * H
    x3 = x_start.reshape(B, R, W)
    n3 = noise.reshape(B, R, W)
    ts = timesteps.astype(jnp.int32)
    ta = jnp.asarray(_SQRT_AC)
    tb = jnp.asarray(_SQRT_OM)

    grid_spec = pltpu.PrefetchScalarGridSpec(
        num_scalar_prefetch=3,
        grid=(1,),
        in_specs=[
            pl.BlockSpec(memory_space=pl.ANY),
            pl.BlockSpec(memory_space=pl.ANY),
        ],
        out_specs=pl.BlockSpec(memory_space=pl.ANY),
        scratch_shapes=[
            pltpu.VMEM((_RING, R, W), jnp.float32),
            pltpu.VMEM((_RING, R, W), jnp.float32),
            pltpu.VMEM((_RING, R, W), jnp.float32),
            pltpu.SemaphoreType.DMA((2 * _RING,)),
            pltpu.SemaphoreType.DMA((_RING,)),
        ],
    )
    out = pl.pallas_call(
        _body,
        grid_spec=grid_spec,
        out_shape=jax.ShapeDtypeStruct((B, R, W), x_start.dtype),
        compiler_params=pltpu.CompilerParams(
            vmem_limit_bytes=40 * 1024 * 1024,
        ),
    )(ts, ta, tb, x3, n3)
    return out.reshape(B, C, H, W)


# speedup vs baseline: 1.3922x; 1.0078x over previous
import math

import jax
import jax.numpy as jnp
import numpy as np
from jax.experimental import pallas as pl
from jax.experimental.pallas import tpu as pltpu

_NUM_TIMESTEPS = 1000


def _schedule_tables():
    steps = _NUM_TIMESTEPS + 1
    x = np.linspace(0, _NUM_TIMESTEPS, steps, dtype=np.float64)
    s = 0.008
    alphas_cumprod = np.cos((x / _NUM_TIMESTEPS + s) / (1 + s) * math.pi * 0.5) ** 2
    alphas_cumprod = alphas_cumprod / alphas_cumprod[0]
    betas = np.clip(1 - alphas_cumprod[1:] / alphas_cumprod[:-1], 0, 0.999)
    ac = np.cumprod(1.0 - betas, axis=0)
    sqrt_ac = np.sqrt(ac).astype(np.float32)
    sqrt_om = np.sqrt(1.0 - ac).astype(np.float32)
    return sqrt_ac, sqrt_om


_SQRT_AC, _SQRT_OM = _schedule_tables()

_NB = 8
_NR = 2


def _body(ts_ref, ta_ref, tb_ref, x_ref, n_ref, o_ref):
    g = pl.program_id(0)
    a = jnp.stack([ta_ref[ts_ref[g * _NB + j]] for j in range(_NB)]).reshape(_NB, 1, 1)
    s = jnp.stack([tb_ref[ts_ref[g * _NB + j]] for j in range(_NB)]).reshape(_NB, 1, 1)
    o_ref[...] = a * x_ref[...] + s * n_ref[...]


def kernel(x_start, noise, timesteps):
    B, C, H, W = x_start.shape
    R = C * H
    x3 = x_start.reshape(B, R, W)
    n3 = noise.reshape(B, R, W)
    ts = timesteps.astype(jnp.int32)
    ta = jnp.asarray(_SQRT_AC)
    tb = jnp.asarray(_SQRT_OM)
    rblk = R // _NR

    grid_spec = pltpu.PrefetchScalarGridSpec(
        num_scalar_prefetch=3,
        grid=(B // _NB, _NR),
        in_specs=[
            pl.BlockSpec((_NB, rblk, W), lambda b, r, *_: (b, r, 0)),
            pl.BlockSpec((_NB, rblk, W), lambda b, r, *_: (b, r, 0)),
        ],
        out_specs=pl.BlockSpec((_NB, rblk, W), lambda b, r, *_: (b, r, 0)),
    )
    out = pl.pallas_call(
        _body,
        grid_spec=grid_spec,
        out_shape=jax.ShapeDtypeStruct((B, R, W), x_start.dtype),
    )(ts, ta, tb, x3, n3)
    return out.reshape(B, C, H, W)


# speedup vs baseline: 1.4226x; 1.0219x over previous
import math

import jax
import jax.numpy as jnp
import numpy as np
from jax.experimental import pallas as pl
from jax.experimental.pallas import tpu as pltpu

_NUM_TIMESTEPS = 1000


def _schedule_tables():
    steps = _NUM_TIMESTEPS + 1
    x = np.linspace(0, _NUM_TIMESTEPS, steps, dtype=np.float64)
    s = 0.008
    alphas_cumprod = np.cos((x / _NUM_TIMESTEPS + s) / (1 + s) * math.pi * 0.5) ** 2
    alphas_cumprod = alphas_cumprod / alphas_cumprod[0]
    betas = np.clip(1 - alphas_cumprod[1:] / alphas_cumprod[:-1], 0, 0.999)
    ac = np.cumprod(1.0 - betas, axis=0)
    sqrt_ac = np.sqrt(ac).astype(np.float32)
    sqrt_om = np.sqrt(1.0 - ac).astype(np.float32)
    return sqrt_ac, sqrt_om


_SQRT_AC, _SQRT_OM = _schedule_tables()

_NB = 8
_NR = 1


def _body(ts_ref, ta_ref, tb_ref, x_ref, n_ref, o_ref):
    g = pl.program_id(0)
    a = jnp.stack([ta_ref[ts_ref[g * _NB + j]] for j in range(_NB)]).reshape(_NB, 1, 1)
    s = jnp.stack([tb_ref[ts_ref[g * _NB + j]] for j in range(_NB)]).reshape(_NB, 1, 1)
    o_ref[...] = a * x_ref[...] + s * n_ref[...]


def kernel(x_start, noise, timesteps):
    B, C, H, W = x_start.shape
    R = C * H
    x3 = x_start.reshape(B, R, W)
    n3 = noise.reshape(B, R, W)
    ts = timesteps.astype(jnp.int32)
    ta = jnp.asarray(_SQRT_AC)
    tb = jnp.asarray(_SQRT_OM)
    rblk = R // _NR

    grid_spec = pltpu.PrefetchScalarGridSpec(
        num_scalar_prefetch=3,
        grid=(B // _NB, _NR),
        in_specs=[
            pl.BlockSpec((_NB, rblk, W), lambda b, r, *_: (b, r, 0)),
            pl.BlockSpec((_NB, rblk, W), lambda b, r, *_: (b, r, 0)),
        ],
        out_specs=pl.BlockSpec((_NB, rblk, W), lambda b, r, *_: (b, r, 0)),
    )
    out = pl.pallas_call(
        _body,
        grid_spec=grid_spec,
        out_shape=jax.ShapeDtypeStruct((B, R, W), x_start.dtype),
    )(ts, ta, tb, x3, n3)
    return out.reshape(B, C, H, W)
